# Initial kernel scaffold; baseline (speedup 1.0000x reference)
import jax
import jax.numpy as jnp
from jax.experimental import pallas as pl


def kernel(features, edge_index, W1, b1, W2, b2, W3, b3):
    raise NotImplementedError("write your pallas kernel here")


# speedup vs baseline: 4.0333x; 4.0333x over previous
import functools

import jax
import jax.numpy as jnp
from jax import lax
from jax.experimental import pallas as pl
from jax.experimental.pallas import tpu as pltpu
from jax.experimental.pallas import tpu_sc as plsc

N = 10000
NPAD = 10240
E = 320000
D = 128
NC = 2
NS = 16
NW = NC * NS
EPW = E // NW
CH = 80
NCHUNK = EPW // CH
RPT = NPAD // NS

_mesh = plsc.VectorSubcoreMesh(core_axis_name="c", subcore_axis_name="s")


@functools.partial(
    pl.kernel,
    out_type=jax.ShapeDtypeStruct((NC, NPAD, D), jnp.float32),
    mesh=_mesh,
    scratch_types=[
        pltpu.VMEM((CH,), jnp.int32),
        pltpu.VMEM((CH,), jnp.int32),
        pltpu.VMEM((CH, D), jnp.float32),
        pltpu.VMEM((CH, D), jnp.float32),
        pltpu.VMEM_SHARED((NPAD, D), jnp.float32),
    ],
)
def _sc_degrees(src, dst, onesa, onesb, zerosd, out,
                idx_s, idx_d, onesa_v, onesb_v, deg):
    c = lax.axis_index("c")
    s = lax.axis_index("s")
    wid = c * NS + s
    r0 = pl.multiple_of(s * RPT, 8)
    e0 = pl.multiple_of(wid * EPW, 8)
    pltpu.sync_copy(zerosd.at[pl.ds(r0, RPT)], deg.at[pl.ds(r0, RPT)])
    pltpu.sync_copy(onesa, onesa_v)
    pltpu.sync_copy(onesb, onesb_v)
    plsc.subcore_barrier()

    def body(j, carry):
        off = pl.multiple_of(e0 + j * CH, 8)
        pltpu.sync_copy(src.at[pl.ds(off, CH)], idx_s)
        pltpu.sync_copy(dst.at[pl.ds(off, CH)], idx_d)
        pltpu.sync_copy(onesa_v, deg.at[idx_s], add=True)
        pltpu.sync_copy(onesb_v, deg.at[idx_d], add=True)
        return carry

    lax.fori_loop(0, NCHUNK, body, 0)
    plsc.subcore_barrier()
    pltpu.sync_copy(deg.at[pl.ds(r0, RPT)], out.at[c, pl.ds(r0, RPT)])


@functools.partial(
    pl.kernel,
    out_type=jax.ShapeDtypeStruct((NC, NPAD, D), jnp.float32),
    mesh=_mesh,
    scratch_types=[
        pltpu.VMEM((CH,), jnp.int32),
        pltpu.VMEM((CH,), jnp.int32),
        pltpu.VMEM((CH, D), jnp.float32),
        pltpu.SemaphoreType.DMA,
        pltpu.VMEM_SHARED((NPAD, D), jnp.float32),
    ],
)
def _sc_propagate(h, src, dst, zerosd, out,
                  idx_s, idx_d, rows, sem, agg):
    c = lax.axis_index("c")
    s = lax.axis_index("s")
    wid = c * NS + s
    r0 = pl.multiple_of(s * RPT, 8)
    e0 = pl.multiple_of(wid * EPW, 8)
    pltpu.sync_copy(zerosd.at[pl.ds(r0, RPT)], agg.at[pl.ds(r0, RPT)])
    plsc.subcore_barrier()

    def body(j, carry):
        off = pl.multiple_of(e0 + j * CH, 8)
        pltpu.sync_copy(src.at[pl.ds(off, CH)], idx_s)
        pltpu.sync_copy(dst.at[pl.ds(off, CH)], idx_d)
        pltpu.async_copy(h.at[idx_s], rows, sem).wait()
        pltpu.sync_copy(rows, agg.at[idx_d], add=True)
        return carry

    lax.fori_loop(0, NCHUNK, body, 0)
    plsc.subcore_barrier()
    pltpu.sync_copy(agg.at[pl.ds(r0, RPT)], out.at[c, pl.ds(r0, RPT)])


BN = 1000


def _tc_prep_body(degp_ref, feat_ref, h1_ref, nsrc_ref, ndst_ref):
    a = degp_ref[...]
    dsrc = a[0, :, 0:1] + a[1, :, 0:1]
    ddst = a[0, :, 1:2] + a[1, :, 1:2]
    ns = lax.rsqrt(jnp.maximum(dsrc, 1.0))
    nd = lax.rsqrt(jnp.maximum(ddst, 1.0))
    h1_ref[...] = feat_ref[...] * ns
    nsrc_ref[...] = ns
    ndst_ref[...] = nd


def _tc_prep(degp, features):
    grid = N // BN
    return pl.pallas_call(
        _tc_prep_body,
        grid=(grid,),
        in_specs=[
            pl.BlockSpec((NC, BN, D), lambda i: (0, i, 0)),
            pl.BlockSpec((BN, D), lambda i: (i, 0)),
        ],
        out_specs=[
            pl.BlockSpec((BN, D), lambda i: (i, 0)),
            pl.BlockSpec((BN, 1), lambda i: (i, 0)),
            pl.BlockSpec((BN, 1), lambda i: (i, 0)),
        ],
        out_shape=[
            jax.ShapeDtypeStruct((N, D), jnp.float32),
            jax.ShapeDtypeStruct((N, 1), jnp.float32),
            jax.ShapeDtypeStruct((N, 1), jnp.float32),
        ],
    )(degp, features)


def _tc_layer_body(aggp_ref, ndst_ref, w_ref, b_ref, nsrc_ref, out_ref, *, last):
    p = (aggp_ref[0] + aggp_ref[1]) * ndst_ref[...]
    y = jnp.dot(p, w_ref[...], preferred_element_type=jnp.float32) + b_ref[...]
    if last:
        out_ref[...] = y
    else:
        out_ref[...] = jnp.maximum(y, 0.0) * nsrc_ref[...]


def _tc_layer(aggp, ndst, w, b, nsrc, last):
    grid = N // BN
    return pl.pallas_call(
        functools.partial(_tc_layer_body, last=last),
        grid=(grid,),
        in_specs=[
            pl.BlockSpec((NC, BN, D), lambda i: (0, i, 0)),
            pl.BlockSpec((BN, 1), lambda i: (i, 0)),
            pl.BlockSpec((D, D), lambda i: (0, 0)),
            pl.BlockSpec((1, D), lambda i: (0, 0)),
            pl.BlockSpec((BN, 1), lambda i: (i, 0)),
        ],
        out_specs=pl.BlockSpec((BN, D), lambda i: (i, 0)),
        out_shape=jax.ShapeDtypeStruct((N, D), jnp.float32),
    )(aggp, ndst, w, b.reshape(1, D), nsrc)


def kernel(features, edge_index, W1, b1, W2, b2, W3, b3):
    src = edge_index[0].astype(jnp.int32)
    dst = edge_index[1].astype(jnp.int32)
    onesa = jnp.zeros((CH, D), jnp.float32).at[:, 0].set(1.0)
    onesb = jnp.zeros((CH, D), jnp.float32).at[:, 1].set(1.0)
    zerosd = jnp.zeros((NPAD, D), jnp.float32)

    degp = _sc_degrees(src, dst, onesa, onesb, zerosd)
    h, nsrc, ndst = _tc_prep(degp, features)
    for w, b, last in ((W1, b1, False), (W2, b2, False), (W3, b3, True)):
        aggp = _sc_propagate(h, src, dst, zerosd)
        h = _tc_layer(aggp, ndst, w, b, nsrc, last)
    return h
